# Initial kernel scaffold; baseline (speedup 1.0000x reference)
import jax
import jax.numpy as jnp
from jax.experimental import pallas as pl


def kernel(edge_index, edge_value, h, w_l0c1, w_l0c2, w_l1c1, gcn_W, gcn_b, lin1_W, lin1_b, lin2_W, lin2_b):
    raise NotImplementedError("write your pallas kernel here")


# speedup vs baseline: 1.0960x; 1.0960x over previous
import functools
import jax
import jax.numpy as jnp
from jax import lax
from jax.experimental import pallas as pl
from jax.experimental.pallas import tpu as pltpu
from jax.experimental.pallas import tpu_sc as plsc

N = 2048
NT = 5
NC = 2
E = 65536
F_IN = 256
F_OUT = 128
NCLS = 8


_SLAB_ROWS = 512
_SLAB_WORDS = _SLAB_ROWS * N
_SHARE = _SLAB_WORDS // 16
_EPT = E // 16
_NVEC = _EPT // 16


def _sc_scatter_body(src_h, dst_h, val_h, out_h,
                     src_v, dst_v, val_v, idx_v, sval_v, zero_v, slab):
    c = lax.axis_index("c")
    s = lax.axis_index("s")

    zf = jnp.zeros((16,), jnp.float32)

    def zinit(i, carry):
        zero_v[pl.ds(i * 16, 16)] = zf
        return carry

    lax.fori_loop(0, 512, zinit, 0)

    def round_body(r, carry):
        e = r // 2
        half = r % 2
        lo = half * 1024 + c * _SLAB_ROWS

        def zcopy(k, cc):
            pltpu.sync_copy(zero_v, slab.at[pl.ds(s * _SHARE + k * 8192, 8192)])
            return cc

        lax.fori_loop(0, 8, zcopy, 0)
        plsc.subcore_barrier()

        pltpu.sync_copy(src_h.at[e, pl.ds(s * _EPT, _EPT)], src_v)
        pltpu.sync_copy(dst_h.at[e, pl.ds(s * _EPT, _EPT)], dst_v)
        pltpu.sync_copy(val_h.at[e, pl.ds(s * _EPT, _EPT)], val_v)

        def vec(i, cc):
            sv = src_v[pl.ds(i * 16, 16)]
            dv = dst_v[pl.ds(i * 16, 16)]
            vv = val_v[pl.ds(i * 16, 16)]
            m = (sv >= lo) & (sv < lo + _SLAB_ROWS)
            iv = jnp.where(m, (sv - lo) * N + dv, 0)
            fv = jnp.where(m, vv, 0.0)
            idx_v[pl.ds(i * 16, 16)] = iv
            sval_v[pl.ds(i * 16, 16)] = fv
            return cc

        lax.fori_loop(0, _NVEC, vec, 0)

        pltpu.sync_copy(sval_v, slab.at[idx_v], add=True)
        plsc.subcore_barrier()

        base = e * (N * N) + lo * N + s * _SHARE
        pltpu.sync_copy(slab.at[pl.ds(s * _SHARE, _SHARE)],
                        out_h.at[pl.ds(base, _SHARE)])
        return carry

    lax.fori_loop(0, NT * 2, round_body, 0)


def _build_adjacency(src, dst, val):
    mesh = plsc.VectorSubcoreMesh(core_axis_name="c", subcore_axis_name="s")
    k = pl.kernel(
        _sc_scatter_body,
        out_type=jax.ShapeDtypeStruct((NT * N * N,), jnp.float32),
        mesh=mesh,
        scratch_types=[
            pltpu.VMEM((_EPT,), jnp.int32),
            pltpu.VMEM((_EPT,), jnp.int32),
            pltpu.VMEM((_EPT,), jnp.float32),
            pltpu.VMEM((_EPT,), jnp.int32),
            pltpu.VMEM((_EPT,), jnp.float32),
            pltpu.VMEM((8192,), jnp.float32),
            pltpu.VMEM_SHARED((_SLAB_WORDS,), jnp.float32),
        ],
    )
    return k(src, dst, val)


_BR = 128


def _combine_body(wa_ref, wb_ref, wc_ref, a_ref, m_ref, u_ref):
    i = pl.program_id(0)
    fa = jax.nn.softmax(wa_ref[...], axis=1)
    fb = jax.nn.softmax(wb_ref[...], axis=1)
    fc = jax.nn.softmax(wc_ref[...], axis=1)
    w = jnp.concatenate([fa, fb, fc], axis=0)
    a = a_ref[...]
    outs = []
    for mchan in range(3 * NC):
        acc = w[mchan, 0] * a[0]
        for t in range(1, NT):
            acc = acc + w[mchan, t] * a[t]
        outs.append(acc)
    m = jnp.stack(outs, axis=0)
    m_ref[...] = m

    @pl.when(i == 0)
    def _():
        u_ref[...] = jnp.zeros_like(u_ref)

    u_ref[...] += m[0:NC].sum(axis=1)[:, None, :]


def _combine(wa, wb, wc, a):
    grid = (N // _BR,)
    return pl.pallas_call(
        _combine_body,
        grid=grid,
        in_specs=[
            pl.BlockSpec((NC, NT), lambda i: (0, 0)),
            pl.BlockSpec((NC, NT), lambda i: (0, 0)),
            pl.BlockSpec((NC, NT), lambda i: (0, 0)),
            pl.BlockSpec((NT, _BR, N), lambda i: (0, i, 0)),
        ],
        out_specs=[
            pl.BlockSpec((3 * NC, _BR, N), lambda i: (0, i, 0)),
            pl.BlockSpec((NC, 1, N), lambda i: (0, 0, 0)),
        ],
        out_shape=[
            jax.ShapeDtypeStruct((3 * NC, N, N), jnp.float32),
            jax.ShapeDtypeStruct((NC, 1, N), jnp.float32),
        ],
        compiler_params=pltpu.CompilerParams(
            dimension_semantics=("arbitrary",)),
    )(wa, wb, wc, a)


_KB = 256


def _mvt_body(prologue, m_ref, v1_ref, v2_ref, o_ref):
    k = pl.program_id(1)
    v = prologue(v1_ref[...], v2_ref[...])[0]
    row = jnp.dot(v, m_ref[0], preferred_element_type=jnp.float32)

    @pl.when(k == 0)
    def _():
        o_ref[...] = jnp.zeros_like(o_ref)

    o_ref[...] += row[None]


def _mvt(mx, base, v1, v2, prologue):
    grid = (NC, N // _KB)
    return pl.pallas_call(
        functools.partial(_mvt_body, prologue),
        grid=grid,
        in_specs=[
            pl.BlockSpec((1, _KB, N), lambda c, k: (base + c, k, 0)),
            pl.BlockSpec((1, 1, _KB), lambda c, k: (c, 0, k)),
            pl.BlockSpec((1, 1, _KB), lambda c, k: (c, 0, k)),
        ],
        out_specs=pl.BlockSpec((1, 1, N), lambda c, k: (c, 0, 0)),
        out_shape=jax.ShapeDtypeStruct((NC, 1, N), jnp.float32),
        compiler_params=pltpu.CompilerParams(
            dimension_semantics=("arbitrary", "arbitrary")),
    )(mx, v1, v2)


def _mvn_body(prologue, m_ref, v1_ref, v2_ref, o_ref):
    v = prologue(v1_ref[...], v2_ref[...])
    o_ref[...] = jnp.dot(m_ref[0], v[0, 0],
                         preferred_element_type=jnp.float32)[None, None, :]


def _mvn(mx, base, v1, v2, prologue):
    grid = (NC, N // _KB)
    return pl.pallas_call(
        functools.partial(_mvn_body, prologue),
        grid=grid,
        in_specs=[
            pl.BlockSpec((1, _KB, N), lambda c, r: (base + c, r, 0)),
            pl.BlockSpec((1, 1, N), lambda c, r: (c, 0, 0)),
            pl.BlockSpec((1, 1, N), lambda c, r: (c, 0, 0)),
        ],
        out_specs=pl.BlockSpec((1, 1, _KB), lambda c, r: (c, 0, r)),
        out_shape=jax.ShapeDtypeStruct((NC, 1, N), jnp.float32),
        compiler_params=pltpu.CompilerParams(
            dimension_semantics=("arbitrary", "arbitrary")),
    )(mx, v1, v2)


def _id2(a, b):
    return a


def _maskf(a, b):
    return (a > 0).astype(jnp.float32)


def _dinv(a, b):
    return jnp.where(a > 0, 1.0 / jnp.where(a > 0, a, 1.0), 0.0)


def _dinv_mul(a, b):
    return jnp.where(a > 0, 1.0 / jnp.where(a > 0, a, 1.0), 0.0) * b


_JB = 256


def _mm1_body(m_ref, h_ref, d_ref, o_ref):
    k = pl.program_id(2)
    d = d_ref[0, 0]
    nout = jnp.where(d > 0, lax.rsqrt(jnp.where(d > 0, d, 1.0)), 0.0)
    hs = h_ref[...] * nout[:, None]
    part = lax.dot_general(m_ref[0], hs, (((0,), (0,)), ((), ())),
                           preferred_element_type=jnp.float32)

    @pl.when(k == 0)
    def _():
        o_ref[...] = jnp.zeros_like(o_ref)

    o_ref[...] += part[None]


def _mm1(mx, h, deg_raw):
    grid = (NC, N // _JB, N // _KB)
    return pl.pallas_call(
        _mm1_body,
        grid=grid,
        in_specs=[
            pl.BlockSpec((1, _KB, _JB), lambda c, j, k: (c, k, j)),
            pl.BlockSpec((_KB, F_IN), lambda c, j, k: (k, 0)),
            pl.BlockSpec((1, 1, _KB), lambda c, j, k: (c, 0, k)),
        ],
        out_specs=pl.BlockSpec((1, _JB, F_IN), lambda c, j, k: (c, j, 0)),
        out_shape=jax.ShapeDtypeStruct((NC, N, F_IN), jnp.float32),
        compiler_params=pltpu.CompilerParams(
            dimension_semantics=("arbitrary", "arbitrary", "arbitrary")),
    )(mx, h, deg_raw)


def _mm2_body(m_ref, t_ref, s_ref, o_ref):
    k = pl.program_id(2)
    nk = pl.num_programs(2)
    part = lax.dot_general(m_ref[0], t_ref[0], (((0,), (0,)), ((), ())),
                           preferred_element_type=jnp.float32)

    @pl.when(k == 0)
    def _():
        o_ref[...] = jnp.zeros_like(o_ref)

    o_ref[...] += part[None]

    @pl.when(k == nk - 1)
    def _():
        sc = _dinv(s_ref[0, 0], None)
        o_ref[...] *= sc[None, :, None]


def _mm2(mx, base, t, s_raw):
    grid = (NC, N // _JB, N // _KB)
    return pl.pallas_call(
        _mm2_body,
        grid=grid,
        in_specs=[
            pl.BlockSpec((1, _KB, _JB), lambda c, j, k: (base + c, k, j)),
            pl.BlockSpec((1, _KB, F_IN), lambda c, j, k: (c, k, 0)),
            pl.BlockSpec((1, 1, _JB), lambda c, j, k: (c, 0, j)),
        ],
        out_specs=pl.BlockSpec((1, _JB, F_IN), lambda c, j, k: (c, j, 0)),
        out_shape=jax.ShapeDtypeStruct((NC, N, F_IN), jnp.float32),
        compiler_params=pltpu.CompilerParams(
            dimension_semantics=("arbitrary", "arbitrary", "arbitrary")),
    )(mx, t, s_raw)


def _mm3_body(m_ref, t_ref, s_ref, w_ref, b_ref, o_ref, acc):
    k = pl.program_id(2)
    nk = pl.num_programs(2)
    part = lax.dot_general(m_ref[0], t_ref[0], (((0,), (0,)), ((), ())),
                           preferred_element_type=jnp.float32)

    @pl.when(k == 0)
    def _():
        acc[...] = jnp.zeros_like(acc)

    acc[...] += part

    @pl.when(k == nk - 1)
    def _():
        sc = _dinv(s_ref[0, 0], None)
        agg = acc[...] * sc[:, None]
        res = jnp.dot(agg, w_ref[...], preferred_element_type=jnp.float32)
        o_ref[...] = jnp.maximum(res + b_ref[...], 0.0)[None]


def _mm3(mx, base, t, s_raw, gcn_w, gcn_b):
    grid = (NC, N // _JB, N // _KB)
    return pl.pallas_call(
        _mm3_body,
        grid=grid,
        in_specs=[
            pl.BlockSpec((1, _KB, _JB), lambda c, j, k: (base + c, k, j)),
            pl.BlockSpec((1, _KB, F_IN), lambda c, j, k: (c, k, 0)),
            pl.BlockSpec((1, 1, _JB), lambda c, j, k: (c, 0, j)),
            pl.BlockSpec((F_IN, F_OUT), lambda c, j, k: (0, 0)),
            pl.BlockSpec((1, F_OUT), lambda c, j, k: (0, 0)),
        ],
        out_specs=pl.BlockSpec((1, _JB, F_OUT), lambda c, j, k: (c, j, 0)),
        out_shape=jax.ShapeDtypeStruct((NC, N, F_OUT), jnp.float32),
        scratch_shapes=[pltpu.VMEM((_JB, F_IN), jnp.float32)],
        compiler_params=pltpu.CompilerParams(
            dimension_semantics=("arbitrary", "arbitrary", "arbitrary")),
    )(mx, t, s_raw, gcn_w, gcn_b)


def _head_body(x_ref, w1_ref, b1_ref, w2_ref, b2_ref, o_ref):
    xc = jnp.concatenate([x_ref[0], x_ref[1]], axis=1)
    x1 = jnp.dot(xc, w1_ref[...], preferred_element_type=jnp.float32)
    x1 = jnp.maximum(x1 + b1_ref[...], 0.0)
    y = jnp.dot(x1, w2_ref[...], preferred_element_type=jnp.float32)
    o_ref[...] = y + b2_ref[...]


def _head(x, w1, b1, w2, b2):
    grid = (N // _JB,)
    return pl.pallas_call(
        _head_body,
        grid=grid,
        in_specs=[
            pl.BlockSpec((NC, _JB, F_OUT), lambda r: (0, r, 0)),
            pl.BlockSpec((NC * F_OUT, F_OUT), lambda r: (0, 0)),
            pl.BlockSpec((1, F_OUT), lambda r: (0, 0)),
            pl.BlockSpec((F_OUT, NCLS), lambda r: (0, 0)),
            pl.BlockSpec((1, NCLS), lambda r: (0, 0)),
        ],
        out_specs=pl.BlockSpec((_JB, NCLS), lambda r: (r, 0)),
        out_shape=jax.ShapeDtypeStruct((N, NCLS), jnp.float32),
        compiler_params=pltpu.CompilerParams(
            dimension_semantics=("arbitrary",)),
    )(x, w1, b1, w2, b2)


def _dense_pipeline(a, h, w_l0c1, w_l0c2, w_l1c1,
                    gcn_W, gcn_b, lin1_W, lin1_b, lin2_W, lin2_b):
    mx, u = _combine(w_l0c1, w_l0c2, w_l1c1, a)

    s0 = _mvt(mx, NC, u, u, _id2)
    s1 = _mvt(mx, 2 * NC, s0, s0, _maskf)
    y1 = _mvn(mx, 2 * NC, s1, s1, _dinv)
    y2 = _mvn(mx, NC, s0, y1, _dinv_mul)
    deg = _mvn(mx, 0, y2, y2, _id2)

    t1 = _mm1(mx, h, deg)
    t2 = _mm2(mx, NC, t1, s0)
    x = _mm3(mx, 2 * NC, t2, s1, gcn_W, gcn_b.reshape(1, F_OUT))
    return _head(x, lin1_W, lin1_b.reshape(1, F_OUT),
                 lin2_W, lin2_b.reshape(1, NCLS))


def kernel(edge_index, edge_value, h, w_l0c1, w_l0c2, w_l1c1,
           gcn_W, gcn_b, lin1_W, lin1_b, lin2_W, lin2_b):
    src = edge_index[:, 0, :].astype(jnp.int32)
    dst = edge_index[:, 1, :].astype(jnp.int32)
    a = _build_adjacency(src, dst, edge_value)
    a = a.reshape(NT, N, N)
    return _dense_pipeline(a, h, w_l0c1, w_l0c2, w_l1c1,
                           gcn_W, gcn_b, lin1_W, lin1_b, lin2_W, lin2_b)


# speedup vs baseline: 1.8062x; 1.6480x over previous
import functools
import jax
import jax.numpy as jnp
from jax import lax
from jax.experimental import pallas as pl
from jax.experimental.pallas import tpu as pltpu
from jax.experimental.pallas import tpu_sc as plsc

N = 2048
NT = 5
NC = 2
E = 65536
F_IN = 256
F_OUT = 128
NCLS = 8


_SLAB_ROWS = 512
_SLAB_WORDS = _SLAB_ROWS * N
_SHARE = _SLAB_WORDS // 16
_EPT = E // 16
_NVEC = _EPT // 16
_CHUNK = 256
_CAP = _EPT + 2 * _CHUNK


def _sc_scatter_body(src_h, dst_h, val_h, out_h,
                     src_v, dst_v, val_v, idx_v, sval_v, zero_v, slab):
    c = lax.axis_index("c")
    s = lax.axis_index("s")

    zf = jnp.zeros((16,), jnp.float32)
    zi = jnp.zeros((16,), jnp.int32)

    def zinit(i, carry):
        zero_v[pl.ds(i * 16, 16)] = zf
        return carry

    lax.fori_loop(0, 512, zinit, 0)

    def round_body(r, carry):
        e = r // 2
        half = r % 2
        lo = half * 1024 + c * _SLAB_ROWS

        def zcopy(k, cc):
            pltpu.sync_copy(zero_v, slab.at[pl.ds(s * _SHARE + k * 8192, 8192)])
            return cc

        lax.fori_loop(0, 8, zcopy, 0)
        plsc.subcore_barrier()

        pltpu.sync_copy(src_h.at[e, pl.ds(s * _EPT, _EPT)], src_v)
        pltpu.sync_copy(dst_h.at[e, pl.ds(s * _EPT, _EPT)], dst_v)
        pltpu.sync_copy(val_h.at[e, pl.ds(s * _EPT, _EPT)], val_v)

        def zpair(i, cc):
            idx_v[pl.ds(i * 16, 16)] = zi
            sval_v[pl.ds(i * 16, 16)] = zf
            return cc

        lax.fori_loop(0, _CAP // 16, zpair, 0)

        def vec(i, nv):
            sv = src_v[pl.ds(i * 16, 16)]
            dv = dst_v[pl.ds(i * 16, 16)]
            vv = val_v[pl.ds(i * 16, 16)]
            m = (sv >= lo) & (sv < lo + _SLAB_ROWS)
            mi = jnp.where(m, jnp.int32(1), jnp.int32(0))
            pref = plsc.cumsum(mi)
            pos = jnp.where(m, nv + pref - 1, _CAP - 16)
            iv = jnp.where(m, (sv - lo) * N + dv, 0)
            fv = jnp.where(m, vv, 0.0)
            plsc.store_scatter(idx_v, [pos], iv)
            plsc.store_scatter(sval_v, [pos], fv)
            return nv + plsc.all_reduce_population_count(m)

        nv = lax.fori_loop(0, _NVEC, vec, jnp.zeros((16,), jnp.int32))
        ns = nv[0]

        def scat(j, cc):
            @pl.when(j * _CHUNK < ns)
            def _():
                pltpu.sync_copy(sval_v.at[pl.ds(j * _CHUNK, _CHUNK)],
                                slab.at[idx_v.at[pl.ds(j * _CHUNK, _CHUNK)]],
                                add=True)
            return cc

        lax.fori_loop(0, _EPT // _CHUNK, scat, 0)
        plsc.subcore_barrier()

        base = e * (N * N) + lo * N + s * _SHARE
        pltpu.sync_copy(slab.at[pl.ds(s * _SHARE, _SHARE)],
                        out_h.at[pl.ds(base, _SHARE)])
        return carry

    lax.fori_loop(0, NT * 2, round_body, 0)


def _build_adjacency(src, dst, val):
    mesh = plsc.VectorSubcoreMesh(core_axis_name="c", subcore_axis_name="s")
    k = pl.kernel(
        _sc_scatter_body,
        out_type=jax.ShapeDtypeStruct((NT * N * N,), jnp.float32),
        mesh=mesh,
        compiler_params=pltpu.CompilerParams(needs_layout_passes=False),
        scratch_types=[
            pltpu.VMEM((_EPT,), jnp.int32),
            pltpu.VMEM((_EPT,), jnp.int32),
            pltpu.VMEM((_EPT,), jnp.float32),
            pltpu.VMEM((_CAP,), jnp.int32),
            pltpu.VMEM((_CAP,), jnp.float32),
            pltpu.VMEM((8192,), jnp.float32),
            pltpu.VMEM_SHARED((_SLAB_WORDS,), jnp.float32),
        ],
    )
    return k(src, dst, val)


_BR = 128


def _combine_body(wa_ref, wb_ref, wc_ref, a_ref, m_ref, u_ref):
    i = pl.program_id(0)
    fa = jax.nn.softmax(wa_ref[...], axis=1)
    fb = jax.nn.softmax(wb_ref[...], axis=1)
    fc = jax.nn.softmax(wc_ref[...], axis=1)
    w = jnp.concatenate([fa, fb, fc], axis=0)
    a = a_ref[...]
    outs = []
    for mchan in range(3 * NC):
        acc = w[mchan, 0] * a[0]
        for t in range(1, NT):
            acc = acc + w[mchan, t] * a[t]
        outs.append(acc)
    m = jnp.stack(outs, axis=0)
    m_ref[...] = m

    @pl.when(i == 0)
    def _():
        u_ref[...] = jnp.zeros_like(u_ref)

    u_ref[...] += m[0:NC].sum(axis=1)[:, None, :]


def _combine(wa, wb, wc, a):
    grid = (N // _BR,)
    return pl.pallas_call(
        _combine_body,
        grid=grid,
        in_specs=[
            pl.BlockSpec((NC, NT), lambda i: (0, 0)),
            pl.BlockSpec((NC, NT), lambda i: (0, 0)),
            pl.BlockSpec((NC, NT), lambda i: (0, 0)),
            pl.BlockSpec((NT, _BR, N), lambda i: (0, i, 0)),
        ],
        out_specs=[
            pl.BlockSpec((3 * NC, _BR, N), lambda i: (0, i, 0)),
            pl.BlockSpec((NC, 1, N), lambda i: (0, 0, 0)),
        ],
        out_shape=[
            jax.ShapeDtypeStruct((3 * NC, N, N), jnp.float32),
            jax.ShapeDtypeStruct((NC, 1, N), jnp.float32),
        ],
        compiler_params=pltpu.CompilerParams(
            dimension_semantics=("arbitrary",)),
    )(wa, wb, wc, a)


_KB = 256


def _mvt_body(prologue, m_ref, v1_ref, v2_ref, o_ref):
    k = pl.program_id(1)
    v = prologue(v1_ref[...], v2_ref[...])[0]
    row = jnp.dot(v, m_ref[0], preferred_element_type=jnp.float32)

    @pl.when(k == 0)
    def _():
        o_ref[...] = jnp.zeros_like(o_ref)

    o_ref[...] += row[None]


def _mvt(mx, base, v1, v2, prologue):
    grid = (NC, N // _KB)
    return pl.pallas_call(
        functools.partial(_mvt_body, prologue),
        grid=grid,
        in_specs=[
            pl.BlockSpec((1, _KB, N), lambda c, k: (base + c, k, 0)),
            pl.BlockSpec((1, 1, _KB), lambda c, k: (c, 0, k)),
            pl.BlockSpec((1, 1, _KB), lambda c, k: (c, 0, k)),
        ],
        out_specs=pl.BlockSpec((1, 1, N), lambda c, k: (c, 0, 0)),
        out_shape=jax.ShapeDtypeStruct((NC, 1, N), jnp.float32),
        compiler_params=pltpu.CompilerParams(
            dimension_semantics=("arbitrary", "arbitrary")),
    )(mx, v1, v2)


def _mvn_body(prologue, m_ref, v1_ref, v2_ref, o_ref):
    v = prologue(v1_ref[...], v2_ref[...])
    o_ref[...] = jnp.dot(m_ref[0], v[0, 0],
                         preferred_element_type=jnp.float32)[None, None, :]


def _mvn(mx, base, v1, v2, prologue):
    grid = (NC, N // _KB)
    return pl.pallas_call(
        functools.partial(_mvn_body, prologue),
        grid=grid,
        in_specs=[
            pl.BlockSpec((1, _KB, N), lambda c, r: (base + c, r, 0)),
            pl.BlockSpec((1, 1, N), lambda c, r: (c, 0, 0)),
            pl.BlockSpec((1, 1, N), lambda c, r: (c, 0, 0)),
        ],
        out_specs=pl.BlockSpec((1, 1, _KB), lambda c, r: (c, 0, r)),
        out_shape=jax.ShapeDtypeStruct((NC, 1, N), jnp.float32),
        compiler_params=pltpu.CompilerParams(
            dimension_semantics=("arbitrary", "arbitrary")),
    )(mx, v1, v2)


def _id2(a, b):
    return a


def _maskf(a, b):
    return (a > 0).astype(jnp.float32)


def _dinv(a, b):
    return jnp.where(a > 0, 1.0 / jnp.where(a > 0, a, 1.0), 0.0)


def _dinv_mul(a, b):
    return jnp.where(a > 0, 1.0 / jnp.where(a > 0, a, 1.0), 0.0) * b


_JB = 256


def _mm1_body(m_ref, h_ref, d_ref, o_ref):
    k = pl.program_id(2)
    d = d_ref[0, 0]
    nout = jnp.where(d > 0, lax.rsqrt(jnp.where(d > 0, d, 1.0)), 0.0)
    hs = h_ref[...] * nout[:, None]
    part = lax.dot_general(m_ref[0], hs, (((0,), (0,)), ((), ())),
                           preferred_element_type=jnp.float32)

    @pl.when(k == 0)
    def _():
        o_ref[...] = jnp.zeros_like(o_ref)

    o_ref[...] += part[None]


def _mm1(mx, h, deg_raw):
    grid = (NC, N // _JB, N // _KB)
    return pl.pallas_call(
        _mm1_body,
        grid=grid,
        in_specs=[
            pl.BlockSpec((1, _KB, _JB), lambda c, j, k: (c, k, j)),
            pl.BlockSpec((_KB, F_IN), lambda c, j, k: (k, 0)),
            pl.BlockSpec((1, 1, _KB), lambda c, j, k: (c, 0, k)),
        ],
        out_specs=pl.BlockSpec((1, _JB, F_IN), lambda c, j, k: (c, j, 0)),
        out_shape=jax.ShapeDtypeStruct((NC, N, F_IN), jnp.float32),
        compiler_params=pltpu.CompilerParams(
            dimension_semantics=("arbitrary", "arbitrary", "arbitrary")),
    )(mx, h, deg_raw)


def _mm2_body(m_ref, t_ref, s_ref, o_ref):
    k = pl.program_id(2)
    nk = pl.num_programs(2)
    part = lax.dot_general(m_ref[0], t_ref[0], (((0,), (0,)), ((), ())),
                           preferred_element_type=jnp.float32)

    @pl.when(k == 0)
    def _():
        o_ref[...] = jnp.zeros_like(o_ref)

    o_ref[...] += part[None]

    @pl.when(k == nk - 1)
    def _():
        sc = _dinv(s_ref[0, 0], None)
        o_ref[...] *= sc[None, :, None]


def _mm2(mx, base, t, s_raw):
    grid = (NC, N // _JB, N // _KB)
    return pl.pallas_call(
        _mm2_body,
        grid=grid,
        in_specs=[
            pl.BlockSpec((1, _KB, _JB), lambda c, j, k: (base + c, k, j)),
            pl.BlockSpec((1, _KB, F_IN), lambda c, j, k: (c, k, 0)),
            pl.BlockSpec((1, 1, _JB), lambda c, j, k: (c, 0, j)),
        ],
        out_specs=pl.BlockSpec((1, _JB, F_IN), lambda c, j, k: (c, j, 0)),
        out_shape=jax.ShapeDtypeStruct((NC, N, F_IN), jnp.float32),
        compiler_params=pltpu.CompilerParams(
            dimension_semantics=("arbitrary", "arbitrary", "arbitrary")),
    )(mx, t, s_raw)


def _mm3_body(m_ref, t_ref, s_ref, w_ref, b_ref, o_ref, acc):
    k = pl.program_id(2)
    nk = pl.num_programs(2)
    part = lax.dot_general(m_ref[0], t_ref[0], (((0,), (0,)), ((), ())),
                           preferred_element_type=jnp.float32)

    @pl.when(k == 0)
    def _():
        acc[...] = jnp.zeros_like(acc)

    acc[...] += part

    @pl.when(k == nk - 1)
    def _():
        sc = _dinv(s_ref[0, 0], None)
        agg = acc[...] * sc[:, None]
        res = jnp.dot(agg, w_ref[...], preferred_element_type=jnp.float32)
        o_ref[...] = jnp.maximum(res + b_ref[...], 0.0)[None]


def _mm3(mx, base, t, s_raw, gcn_w, gcn_b):
    grid = (NC, N // _JB, N // _KB)
    return pl.pallas_call(
        _mm3_body,
        grid=grid,
        in_specs=[
            pl.BlockSpec((1, _KB, _JB), lambda c, j, k: (base + c, k, j)),
            pl.BlockSpec((1, _KB, F_IN), lambda c, j, k: (c, k, 0)),
            pl.BlockSpec((1, 1, _JB), lambda c, j, k: (c, 0, j)),
            pl.BlockSpec((F_IN, F_OUT), lambda c, j, k: (0, 0)),
            pl.BlockSpec((1, F_OUT), lambda c, j, k: (0, 0)),
        ],
        out_specs=pl.BlockSpec((1, _JB, F_OUT), lambda c, j, k: (c, j, 0)),
        out_shape=jax.ShapeDtypeStruct((NC, N, F_OUT), jnp.float32),
        scratch_shapes=[pltpu.VMEM((_JB, F_IN), jnp.float32)],
        compiler_params=pltpu.CompilerParams(
            dimension_semantics=("arbitrary", "arbitrary", "arbitrary")),
    )(mx, t, s_raw, gcn_w, gcn_b)


def _head_body(x_ref, w1_ref, b1_ref, w2_ref, b2_ref, o_ref):
    xc = jnp.concatenate([x_ref[0], x_ref[1]], axis=1)
    x1 = jnp.dot(xc, w1_ref[...], preferred_element_type=jnp.float32)
    x1 = jnp.maximum(x1 + b1_ref[...], 0.0)
    y = jnp.dot(x1, w2_ref[...], preferred_element_type=jnp.float32)
    o_ref[...] = y + b2_ref[...]


def _head(x, w1, b1, w2, b2):
    grid = (N // _JB,)
    return pl.pallas_call(
        _head_body,
        grid=grid,
        in_specs=[
            pl.BlockSpec((NC, _JB, F_OUT), lambda r: (0, r, 0)),
            pl.BlockSpec((NC * F_OUT, F_OUT), lambda r: (0, 0)),
            pl.BlockSpec((1, F_OUT), lambda r: (0, 0)),
            pl.BlockSpec((F_OUT, NCLS), lambda r: (0, 0)),
            pl.BlockSpec((1, NCLS), lambda r: (0, 0)),
        ],
        out_specs=pl.BlockSpec((_JB, NCLS), lambda r: (r, 0)),
        out_shape=jax.ShapeDtypeStruct((N, NCLS), jnp.float32),
        compiler_params=pltpu.CompilerParams(
            dimension_semantics=("arbitrary",)),
    )(x, w1, b1, w2, b2)


def _dense_pipeline(a, h, w_l0c1, w_l0c2, w_l1c1,
                    gcn_W, gcn_b, lin1_W, lin1_b, lin2_W, lin2_b):
    mx, u = _combine(w_l0c1, w_l0c2, w_l1c1, a)

    s0 = _mvt(mx, NC, u, u, _id2)
    s1 = _mvt(mx, 2 * NC, s0, s0, _maskf)
    y1 = _mvn(mx, 2 * NC, s1, s1, _dinv)
    y2 = _mvn(mx, NC, s0, y1, _dinv_mul)
    deg = _mvn(mx, 0, y2, y2, _id2)

    t1 = _mm1(mx, h, deg)
    t2 = _mm2(mx, NC, t1, s0)
    x = _mm3(mx, 2 * NC, t2, s1, gcn_W, gcn_b.reshape(1, F_OUT))
    return _head(x, lin1_W, lin1_b.reshape(1, F_OUT),
                 lin2_W, lin2_b.reshape(1, NCLS))


def kernel(edge_index, edge_value, h, w_l0c1, w_l0c2, w_l1c1,
           gcn_W, gcn_b, lin1_W, lin1_b, lin2_W, lin2_b):
    src = edge_index[:, 0, :].astype(jnp.int32)
    dst = edge_index[:, 1, :].astype(jnp.int32)
    a = _build_adjacency(src, dst, edge_value)
    a = a.reshape(NT, N, N)
    return _dense_pipeline(a, h, w_l0c1, w_l0c2, w_l1c1,
                           gcn_W, gcn_b, lin1_W, lin1_b, lin2_W, lin2_b)


# speedup vs baseline: 2.4683x; 1.3666x over previous
import functools
import jax
import jax.numpy as jnp
from jax import lax
from jax.experimental import pallas as pl
from jax.experimental.pallas import tpu as pltpu
from jax.experimental.pallas import tpu_sc as plsc

N = 2048
NT = 5
NC = 2
E = 65536
F_IN = 256
F_OUT = 128
NCLS = 8


_SLAB_ROWS = 512
_SLAB_WORDS = _SLAB_ROWS * N
_SHARE = _SLAB_WORDS // 16
_EPT = E // 16
_NVEC = _EPT // 16
_CHUNK = 256
_CAP = _EPT + 2 * _CHUNK


def _sc_scatter_body(src_h, dst_h, val_h, out_h,
                     src_v, dst_v, val_v, idx_v, sval_v, zero_v, slab):
    c = lax.axis_index("c")
    s = lax.axis_index("s")

    zf = jnp.zeros((16,), jnp.float32)
    zi = jnp.zeros((16,), jnp.int32)

    def zinit(i, carry):
        zero_v[pl.ds(i * 16, 16)] = zf
        return carry

    lax.fori_loop(0, 512, zinit, 0)

    def round_body(r, carry):
        e = r // 2
        half = r % 2
        lo = half * 1024 + c * _SLAB_ROWS

        def zcopy(k, cc):
            pltpu.sync_copy(zero_v, slab.at[pl.ds(s * _SHARE + k * 8192, 8192)])
            return cc

        lax.fori_loop(0, 8, zcopy, 0)
        plsc.subcore_barrier()

        pltpu.sync_copy(src_h.at[e, pl.ds(s * _EPT, _EPT)], src_v)
        pltpu.sync_copy(dst_h.at[e, pl.ds(s * _EPT, _EPT)], dst_v)
        pltpu.sync_copy(val_h.at[e, pl.ds(s * _EPT, _EPT)], val_v)

        def zpair(i, cc):
            idx_v[pl.ds(i * 16, 16)] = zi
            sval_v[pl.ds(i * 16, 16)] = zf
            return cc

        lax.fori_loop(0, _CAP // 16, zpair, 0)

        def vec(i, nv):
            sv = src_v[pl.ds(i * 16, 16)]
            dv = dst_v[pl.ds(i * 16, 16)]
            vv = val_v[pl.ds(i * 16, 16)]
            m = (sv >= lo) & (sv < lo + _SLAB_ROWS)
            mi = jnp.where(m, jnp.int32(1), jnp.int32(0))
            pref = plsc.cumsum(mi)
            pos = jnp.where(m, nv + pref - 1, _CAP - 16)
            iv = jnp.where(m, (sv - lo) * N + dv, 0)
            fv = jnp.where(m, vv, 0.0)
            plsc.store_scatter(idx_v, [pos], iv)
            plsc.store_scatter(sval_v, [pos], fv)
            return nv + plsc.all_reduce_population_count(m)

        nv = lax.fori_loop(0, _NVEC, vec, jnp.zeros((16,), jnp.int32))
        ns = nv[0]

        def scat(j, cc):
            @pl.when(j * _CHUNK < ns)
            def _():
                pltpu.sync_copy(sval_v.at[pl.ds(j * _CHUNK, _CHUNK)],
                                slab.at[idx_v.at[pl.ds(j * _CHUNK, _CHUNK)]],
                                add=True)
            return cc

        lax.fori_loop(0, _EPT // _CHUNK, scat, 0)
        plsc.subcore_barrier()

        base = e * (N * N) + lo * N + s * _SHARE
        pltpu.sync_copy(slab.at[pl.ds(s * _SHARE, _SHARE)],
                        out_h.at[pl.ds(base, _SHARE)])
        return carry

    lax.fori_loop(0, NT * 2, round_body, 0)


def _build_adjacency(src, dst, val):
    mesh = plsc.VectorSubcoreMesh(core_axis_name="c", subcore_axis_name="s")
    k = pl.kernel(
        _sc_scatter_body,
        out_type=jax.ShapeDtypeStruct((NT * N * N,), jnp.float32),
        mesh=mesh,
        compiler_params=pltpu.CompilerParams(needs_layout_passes=False),
        scratch_types=[
            pltpu.VMEM((_EPT,), jnp.int32),
            pltpu.VMEM((_EPT,), jnp.int32),
            pltpu.VMEM((_EPT,), jnp.float32),
            pltpu.VMEM((_CAP,), jnp.int32),
            pltpu.VMEM((_CAP,), jnp.float32),
            pltpu.VMEM((8192,), jnp.float32),
            pltpu.VMEM_SHARED((_SLAB_WORDS,), jnp.float32),
        ],
    )
    return k(src, dst, val)


_BR = 128


def _combine_body(wa_ref, wb_ref, wc_ref, a_ref, m_ref, u_ref):
    i = pl.program_id(0)
    fa = jax.nn.softmax(wa_ref[...], axis=1)
    fb = jax.nn.softmax(wb_ref[...], axis=1)
    fc = jax.nn.softmax(wc_ref[...], axis=1)
    w = jnp.concatenate([fa, fb, fc], axis=0)
    a = a_ref[...]
    outs = []
    for mchan in range(3 * NC):
        acc = w[mchan, 0] * a[0]
        for t in range(1, NT):
            acc = acc + w[mchan, t] * a[t]
        outs.append(acc)
    m = jnp.stack(outs, axis=0)
    m_ref[...] = m.astype(jnp.bfloat16)

    @pl.when(i == 0)
    def _():
        u_ref[...] = jnp.zeros_like(u_ref)

    u_ref[...] += m[0:NC].sum(axis=1)[:, None, :]


def _combine(wa, wb, wc, a):
    grid = (N // _BR,)
    return pl.pallas_call(
        _combine_body,
        grid=grid,
        in_specs=[
            pl.BlockSpec((NC, NT), lambda i: (0, 0)),
            pl.BlockSpec((NC, NT), lambda i: (0, 0)),
            pl.BlockSpec((NC, NT), lambda i: (0, 0)),
            pl.BlockSpec((NT, _BR, N), lambda i: (0, i, 0)),
        ],
        out_specs=[
            pl.BlockSpec((3 * NC, _BR, N), lambda i: (0, i, 0)),
            pl.BlockSpec((NC, 1, N), lambda i: (0, 0, 0)),
        ],
        out_shape=[
            jax.ShapeDtypeStruct((3 * NC, N, N), jnp.bfloat16),
            jax.ShapeDtypeStruct((NC, 1, N), jnp.float32),
        ],
        compiler_params=pltpu.CompilerParams(
            dimension_semantics=("arbitrary",)),
    )(wa, wb, wc, a)


_KB = 256


def _mvt_body(prologue, m_ref, v1_ref, v2_ref, o_ref):
    k = pl.program_id(1)
    v = prologue(v1_ref[...], v2_ref[...])[0].astype(jnp.bfloat16)
    row = jnp.dot(v, m_ref[0], preferred_element_type=jnp.float32)

    @pl.when(k == 0)
    def _():
        o_ref[...] = jnp.zeros_like(o_ref)

    o_ref[...] += row[None]


def _mvt(mx, base, v1, v2, prologue):
    grid = (NC, N // _KB)
    return pl.pallas_call(
        functools.partial(_mvt_body, prologue),
        grid=grid,
        in_specs=[
            pl.BlockSpec((1, _KB, N), lambda c, k: (base + c, k, 0)),
            pl.BlockSpec((1, 1, _KB), lambda c, k: (c, 0, k)),
            pl.BlockSpec((1, 1, _KB), lambda c, k: (c, 0, k)),
        ],
        out_specs=pl.BlockSpec((1, 1, N), lambda c, k: (c, 0, 0)),
        out_shape=jax.ShapeDtypeStruct((NC, 1, N), jnp.float32),
        compiler_params=pltpu.CompilerParams(
            dimension_semantics=("arbitrary", "arbitrary")),
    )(mx, v1, v2)


def _mvn_body(prologue, m_ref, v1_ref, v2_ref, o_ref):
    v = prologue(v1_ref[...], v2_ref[...])[0, 0].astype(jnp.bfloat16)
    o_ref[...] = jnp.dot(m_ref[0], v,
                         preferred_element_type=jnp.float32)[None, None, :]


def _mvn(mx, base, v1, v2, prologue):
    grid = (NC, N // _KB)
    return pl.pallas_call(
        functools.partial(_mvn_body, prologue),
        grid=grid,
        in_specs=[
            pl.BlockSpec((1, _KB, N), lambda c, r: (base + c, r, 0)),
            pl.BlockSpec((1, 1, N), lambda c, r: (c, 0, 0)),
            pl.BlockSpec((1, 1, N), lambda c, r: (c, 0, 0)),
        ],
        out_specs=pl.BlockSpec((1, 1, _KB), lambda c, r: (c, 0, r)),
        out_shape=jax.ShapeDtypeStruct((NC, 1, N), jnp.float32),
        compiler_params=pltpu.CompilerParams(
            dimension_semantics=("arbitrary", "arbitrary")),
    )(mx, v1, v2)


def _id2(a, b):
    return a


def _maskf(a, b):
    return (a > 0).astype(jnp.float32)


def _dinv(a, b):
    return jnp.where(a > 0, 1.0 / jnp.where(a > 0, a, 1.0), 0.0)


def _dinv_mul(a, b):
    return jnp.where(a > 0, 1.0 / jnp.where(a > 0, a, 1.0), 0.0) * b


_JB = 1024


def _mm1_body(m_ref, h_ref, d_ref, o_ref):
    k = pl.program_id(2)
    d = d_ref[0, 0]
    nout = jnp.where(d > 0, lax.rsqrt(jnp.where(d > 0, d, 1.0)), 0.0)
    hs = (h_ref[...] * nout[:, None]).astype(jnp.bfloat16)
    part = lax.dot_general(m_ref[0], hs, (((0,), (0,)), ((), ())),
                           preferred_element_type=jnp.float32)

    @pl.when(k == 0)
    def _():
        o_ref[...] = jnp.zeros_like(o_ref)

    o_ref[...] += part[None]


def _mm1(mx, h, deg_raw):
    grid = (NC, N // _JB, N // _KB)
    return pl.pallas_call(
        _mm1_body,
        grid=grid,
        in_specs=[
            pl.BlockSpec((1, _KB, _JB), lambda c, j, k: (c, k, j)),
            pl.BlockSpec((_KB, F_IN), lambda c, j, k: (k, 0)),
            pl.BlockSpec((1, 1, _KB), lambda c, j, k: (c, 0, k)),
        ],
        out_specs=pl.BlockSpec((1, _JB, F_IN), lambda c, j, k: (c, j, 0)),
        out_shape=jax.ShapeDtypeStruct((NC, N, F_IN), jnp.float32),
        compiler_params=pltpu.CompilerParams(
            dimension_semantics=("arbitrary", "arbitrary", "arbitrary")),
    )(mx, h, deg_raw)


def _mm2_body(m_ref, t_ref, s_ref, o_ref):
    k = pl.program_id(2)
    nk = pl.num_programs(2)
    part = lax.dot_general(m_ref[0], t_ref[0].astype(jnp.bfloat16),
                           (((0,), (0,)), ((), ())),
                           preferred_element_type=jnp.float32)

    @pl.when(k == 0)
    def _():
        o_ref[...] = jnp.zeros_like(o_ref)

    o_ref[...] += part[None]

    @pl.when(k == nk - 1)
    def _():
        sc = _dinv(s_ref[0, 0], None)
        o_ref[...] *= sc[None, :, None]


def _mm2(mx, base, t, s_raw):
    grid = (NC, N // _JB, N // _KB)
    return pl.pallas_call(
        _mm2_body,
        grid=grid,
        in_specs=[
            pl.BlockSpec((1, _KB, _JB), lambda c, j, k: (base + c, k, j)),
            pl.BlockSpec((1, _KB, F_IN), lambda c, j, k: (c, k, 0)),
            pl.BlockSpec((1, 1, _JB), lambda c, j, k: (c, 0, j)),
        ],
        out_specs=pl.BlockSpec((1, _JB, F_IN), lambda c, j, k: (c, j, 0)),
        out_shape=jax.ShapeDtypeStruct((NC, N, F_IN), jnp.float32),
        compiler_params=pltpu.CompilerParams(
            dimension_semantics=("arbitrary", "arbitrary", "arbitrary")),
    )(mx, t, s_raw)


def _mm3_body(m_ref, t_ref, s_ref, w_ref, b_ref, o_ref, acc):
    k = pl.program_id(2)
    nk = pl.num_programs(2)
    part = lax.dot_general(m_ref[0], t_ref[0].astype(jnp.bfloat16),
                           (((0,), (0,)), ((), ())),
                           preferred_element_type=jnp.float32)

    @pl.when(k == 0)
    def _():
        acc[...] = jnp.zeros_like(acc)

    acc[...] += part

    @pl.when(k == nk - 1)
    def _():
        sc = _dinv(s_ref[0, 0], None)
        agg = acc[...] * sc[:, None]
        res = jnp.dot(agg, w_ref[...], preferred_element_type=jnp.float32)
        o_ref[...] = jnp.maximum(res + b_ref[...], 0.0)[None]


def _mm3(mx, base, t, s_raw, gcn_w, gcn_b):
    grid = (NC, N // _JB, N // _KB)
    return pl.pallas_call(
        _mm3_body,
        grid=grid,
        in_specs=[
            pl.BlockSpec((1, _KB, _JB), lambda c, j, k: (base + c, k, j)),
            pl.BlockSpec((1, _KB, F_IN), lambda c, j, k: (c, k, 0)),
            pl.BlockSpec((1, 1, _JB), lambda c, j, k: (c, 0, j)),
            pl.BlockSpec((F_IN, F_OUT), lambda c, j, k: (0, 0)),
            pl.BlockSpec((1, F_OUT), lambda c, j, k: (0, 0)),
        ],
        out_specs=pl.BlockSpec((1, _JB, F_OUT), lambda c, j, k: (c, j, 0)),
        out_shape=jax.ShapeDtypeStruct((NC, N, F_OUT), jnp.float32),
        scratch_shapes=[pltpu.VMEM((_JB, F_IN), jnp.float32)],
        compiler_params=pltpu.CompilerParams(
            dimension_semantics=("arbitrary", "arbitrary", "arbitrary")),
    )(mx, t, s_raw, gcn_w, gcn_b)


def _head_body(x_ref, w1_ref, b1_ref, w2_ref, b2_ref, o_ref):
    xc = jnp.concatenate([x_ref[0], x_ref[1]], axis=1)
    x1 = jnp.dot(xc, w1_ref[...], preferred_element_type=jnp.float32)
    x1 = jnp.maximum(x1 + b1_ref[...], 0.0)
    y = jnp.dot(x1, w2_ref[...], preferred_element_type=jnp.float32)
    o_ref[...] = y + b2_ref[...]


def _head(x, w1, b1, w2, b2):
    grid = (N // _JB,)
    return pl.pallas_call(
        _head_body,
        grid=grid,
        in_specs=[
            pl.BlockSpec((NC, _JB, F_OUT), lambda r: (0, r, 0)),
            pl.BlockSpec((NC * F_OUT, F_OUT), lambda r: (0, 0)),
            pl.BlockSpec((1, F_OUT), lambda r: (0, 0)),
            pl.BlockSpec((F_OUT, NCLS), lambda r: (0, 0)),
            pl.BlockSpec((1, NCLS), lambda r: (0, 0)),
        ],
        out_specs=pl.BlockSpec((_JB, NCLS), lambda r: (r, 0)),
        out_shape=jax.ShapeDtypeStruct((N, NCLS), jnp.float32),
        compiler_params=pltpu.CompilerParams(
            dimension_semantics=("arbitrary",)),
    )(x, w1, b1, w2, b2)


def _dense_pipeline(a, h, w_l0c1, w_l0c2, w_l1c1,
                    gcn_W, gcn_b, lin1_W, lin1_b, lin2_W, lin2_b):
    mx, u = _combine(w_l0c1, w_l0c2, w_l1c1, a)

    s0 = _mvt(mx, NC, u, u, _id2)
    s1 = _mvt(mx, 2 * NC, s0, s0, _maskf)
    y1 = _mvn(mx, 2 * NC, s1, s1, _dinv)
    y2 = _mvn(mx, NC, s0, y1, _dinv_mul)
    deg = _mvn(mx, 0, y2, y2, _id2)

    t1 = _mm1(mx, h, deg)
    t2 = _mm2(mx, NC, t1, s0)
    x = _mm3(mx, 2 * NC, t2, s1, gcn_W, gcn_b.reshape(1, F_OUT))
    return _head(x, lin1_W, lin1_b.reshape(1, F_OUT),
                 lin2_W, lin2_b.reshape(1, NCLS))


def kernel(edge_index, edge_value, h, w_l0c1, w_l0c2, w_l1c1,
           gcn_W, gcn_b, lin1_W, lin1_b, lin2_W, lin2_b):
    src = edge_index[:, 0, :].astype(jnp.int32)
    dst = edge_index[:, 1, :].astype(jnp.int32)
    a = _build_adjacency(src, dst, edge_value)
    a = a.reshape(NT, N, N)
    return _dense_pipeline(a, h, w_l0c1, w_l0c2, w_l1c1,
                           gcn_W, gcn_b, lin1_W, lin1_b, lin2_W, lin2_b)


# speedup vs baseline: 2.6351x; 1.0676x over previous
import functools
import jax
import jax.numpy as jnp
from jax import lax
from jax.experimental import pallas as pl
from jax.experimental.pallas import tpu as pltpu
from jax.experimental.pallas import tpu_sc as plsc

N = 2048
NT = 5
NC = 2
E = 65536
F_IN = 256
F_OUT = 128
NCLS = 8


_SLAB_ROWS = 512
_SLAB_WORDS = _SLAB_ROWS * N
_SHARE = _SLAB_WORDS // 16
_EPT = E // 16
_NVEC = _EPT // 16
_CHUNK = 256
_CAP = _EPT + 2 * _CHUNK


def _sc_scatter_body(src_h, dst_h, val_h, zeros_h, out_h,
                     src_v, dst_v, val_v, idx_v, sval_v, sem_e, sem_z, sem_s,
                     slab):
    c = lax.axis_index("c")
    s = lax.axis_index("s")

    zf = jnp.zeros((16,), jnp.float32)
    zi = jnp.zeros((16,), jnp.int32)

    def round_body(r, carry):
        e = r // 2
        half = r % 2
        lo = half * 1024 + c * _SLAB_ROWS

        zcp = pltpu.async_copy(zeros_h.at[pl.ds(s * _SHARE, _SHARE)],
                               slab.at[pl.ds(s * _SHARE, _SHARE)], sem_z)
        cp1 = pltpu.async_copy(src_h.at[e, pl.ds(s * _EPT, _EPT)], src_v, sem_e)
        cp2 = pltpu.async_copy(dst_h.at[e, pl.ds(s * _EPT, _EPT)], dst_v, sem_e)
        cp3 = pltpu.async_copy(val_h.at[e, pl.ds(s * _EPT, _EPT)], val_v, sem_e)
        cp1.wait()
        cp2.wait()
        cp3.wait()

        def zpair(i, cc):
            idx_v[pl.ds(i * 16, 16)] = zi
            sval_v[pl.ds(i * 16, 16)] = zf
            return cc

        lax.fori_loop(0, _CAP // 16, zpair, 0)

        def vec(i, nv):
            sv = src_v[pl.ds(i * 16, 16)]
            dv = dst_v[pl.ds(i * 16, 16)]
            vv = val_v[pl.ds(i * 16, 16)]
            m = (sv >= lo) & (sv < lo + _SLAB_ROWS)
            mi = jnp.where(m, jnp.int32(1), jnp.int32(0))
            pref = plsc.cumsum(mi)
            pos = jnp.where(m, nv + pref - 1, _CAP - 16)
            iv = jnp.where(m, (sv - lo) * N + dv, 0)
            fv = jnp.where(m, vv, 0.0)
            plsc.store_scatter(idx_v, [pos], iv)
            plsc.store_scatter(sval_v, [pos], fv)
            return nv + plsc.all_reduce_population_count(m)

        nv = lax.fori_loop(0, _NVEC, vec, jnp.zeros((16,), jnp.int32))
        ns = nv[0]

        zcp.wait()
        plsc.subcore_barrier()

        for j in range(_EPT // _CHUNK):
            @pl.when(j * _CHUNK < ns)
            def _(j=j):
                pltpu.async_copy(sval_v.at[pl.ds(j * _CHUNK, _CHUNK)],
                                 slab.at[idx_v.at[pl.ds(j * _CHUNK, _CHUNK)]],
                                 sem_s, add=True)

        for j in range(_EPT // _CHUNK):
            @pl.when(j * _CHUNK < ns)
            def _(j=j):
                pltpu.make_async_copy(
                    sval_v.at[pl.ds(j * _CHUNK, _CHUNK)],
                    slab.at[idx_v.at[pl.ds(j * _CHUNK, _CHUNK)]],
                    sem_s).wait()
        plsc.subcore_barrier()

        base = e * (N * N) + lo * N + s * _SHARE
        pltpu.sync_copy(slab.at[pl.ds(s * _SHARE, _SHARE)],
                        out_h.at[pl.ds(base, _SHARE)])
        return carry

    lax.fori_loop(0, NT * 2, round_body, 0)


def _build_adjacency(src, dst, val):
    mesh = plsc.VectorSubcoreMesh(core_axis_name="c", subcore_axis_name="s")
    k = pl.kernel(
        _sc_scatter_body,
        out_type=jax.ShapeDtypeStruct((NT * N * N,), jnp.float32),
        mesh=mesh,
        compiler_params=pltpu.CompilerParams(needs_layout_passes=False),
        scratch_types=[
            pltpu.VMEM((_EPT,), jnp.int32),
            pltpu.VMEM((_EPT,), jnp.int32),
            pltpu.VMEM((_EPT,), jnp.float32),
            pltpu.VMEM((_CAP,), jnp.int32),
            pltpu.VMEM((_CAP,), jnp.float32),
            pltpu.SemaphoreType.DMA,
            pltpu.SemaphoreType.DMA,
            pltpu.SemaphoreType.DMA,
            pltpu.VMEM_SHARED((_SLAB_WORDS,), jnp.float32),
        ],
    )
    zeros = jnp.zeros((_SLAB_WORDS,), jnp.float32)
    return k(src, dst, val, zeros)


_BR = 128


def _combine_body(wa_ref, wb_ref, wc_ref, a_ref, m_ref, u_ref):
    i = pl.program_id(0)
    fa = jax.nn.softmax(wa_ref[...], axis=1)
    fb = jax.nn.softmax(wb_ref[...], axis=1)
    fc = jax.nn.softmax(wc_ref[...], axis=1)
    w = jnp.concatenate([fa, fb, fc], axis=0)
    a = a_ref[...]
    outs = []
    for mchan in range(3 * NC):
        acc = w[mchan, 0] * a[0]
        for t in range(1, NT):
            acc = acc + w[mchan, t] * a[t]
        outs.append(acc)
    m = jnp.stack(outs, axis=0)
    m_ref[...] = m.astype(jnp.bfloat16)

    @pl.when(i == 0)
    def _():
        u_ref[...] = jnp.zeros_like(u_ref)

    u_ref[...] += m[0:NC].sum(axis=1)[:, None, :]


def _combine(wa, wb, wc, a):
    grid = (N // _BR,)
    return pl.pallas_call(
        _combine_body,
        grid=grid,
        in_specs=[
            pl.BlockSpec((NC, NT), lambda i: (0, 0)),
            pl.BlockSpec((NC, NT), lambda i: (0, 0)),
            pl.BlockSpec((NC, NT), lambda i: (0, 0)),
            pl.BlockSpec((NT, _BR, N), lambda i: (0, i, 0)),
        ],
        out_specs=[
            pl.BlockSpec((3 * NC, _BR, N), lambda i: (0, i, 0)),
            pl.BlockSpec((NC, 1, N), lambda i: (0, 0, 0)),
        ],
        out_shape=[
            jax.ShapeDtypeStruct((3 * NC, N, N), jnp.bfloat16),
            jax.ShapeDtypeStruct((NC, 1, N), jnp.float32),
        ],
        compiler_params=pltpu.CompilerParams(
            dimension_semantics=("arbitrary",)),
    )(wa, wb, wc, a)


_KB = 256


def _mvt_body(prologue, m_ref, v1_ref, v2_ref, o_ref):
    k = pl.program_id(1)
    v = prologue(v1_ref[...], v2_ref[...])[0].astype(jnp.bfloat16)
    row = jnp.dot(v, m_ref[0], preferred_element_type=jnp.float32)

    @pl.when(k == 0)
    def _():
        o_ref[...] = jnp.zeros_like(o_ref)

    o_ref[...] += row[None]


def _mvt(mx, base, v1, v2, prologue):
    grid = (NC, N // _KB)
    return pl.pallas_call(
        functools.partial(_mvt_body, prologue),
        grid=grid,
        in_specs=[
            pl.BlockSpec((1, _KB, N), lambda c, k: (base + c, k, 0)),
            pl.BlockSpec((1, 1, _KB), lambda c, k: (c, 0, k)),
            pl.BlockSpec((1, 1, _KB), lambda c, k: (c, 0, k)),
        ],
        out_specs=pl.BlockSpec((1, 1, N), lambda c, k: (c, 0, 0)),
        out_shape=jax.ShapeDtypeStruct((NC, 1, N), jnp.float32),
        compiler_params=pltpu.CompilerParams(
            dimension_semantics=("arbitrary", "arbitrary")),
    )(mx, v1, v2)


def _mvn_body(prologue, m_ref, v1_ref, v2_ref, o_ref):
    v = prologue(v1_ref[...], v2_ref[...])[0, 0].astype(jnp.bfloat16)
    o_ref[...] = jnp.dot(m_ref[0], v,
                         preferred_element_type=jnp.float32)[None, None, :]


def _mvn(mx, base, v1, v2, prologue):
    grid = (NC, N // _KB)
    return pl.pallas_call(
        functools.partial(_mvn_body, prologue),
        grid=grid,
        in_specs=[
            pl.BlockSpec((1, _KB, N), lambda c, r: (base + c, r, 0)),
            pl.BlockSpec((1, 1, N), lambda c, r: (c, 0, 0)),
            pl.BlockSpec((1, 1, N), lambda c, r: (c, 0, 0)),
        ],
        out_specs=pl.BlockSpec((1, 1, _KB), lambda c, r: (c, 0, r)),
        out_shape=jax.ShapeDtypeStruct((NC, 1, N), jnp.float32),
        compiler_params=pltpu.CompilerParams(
            dimension_semantics=("arbitrary", "arbitrary")),
    )(mx, v1, v2)


def _id2(a, b):
    return a


def _maskf(a, b):
    return (a > 0).astype(jnp.float32)


def _dinv(a, b):
    return jnp.where(a > 0, 1.0 / jnp.where(a > 0, a, 1.0), 0.0)


def _dinv_mul(a, b):
    return jnp.where(a > 0, 1.0 / jnp.where(a > 0, a, 1.0), 0.0) * b


_JB = 1024


def _mm1_body(m_ref, h_ref, d_ref, o_ref):
    k = pl.program_id(2)
    d = d_ref[0, 0]
    nout = jnp.where(d > 0, lax.rsqrt(jnp.where(d > 0, d, 1.0)), 0.0)
    hs = (h_ref[...] * nout[:, None]).astype(jnp.bfloat16)
    part = lax.dot_general(m_ref[0], hs, (((0,), (0,)), ((), ())),
                           preferred_element_type=jnp.float32)

    @pl.when(k == 0)
    def _():
        o_ref[...] = jnp.zeros_like(o_ref)

    o_ref[...] += part[None]


def _mm1(mx, h, deg_raw):
    grid = (NC, N // _JB, N // _KB)
    return pl.pallas_call(
        _mm1_body,
        grid=grid,
        in_specs=[
            pl.BlockSpec((1, _KB, _JB), lambda c, j, k: (c, k, j)),
            pl.BlockSpec((_KB, F_IN), lambda c, j, k: (k, 0)),
            pl.BlockSpec((1, 1, _KB), lambda c, j, k: (c, 0, k)),
        ],
        out_specs=pl.BlockSpec((1, _JB, F_IN), lambda c, j, k: (c, j, 0)),
        out_shape=jax.ShapeDtypeStruct((NC, N, F_IN), jnp.float32),
        compiler_params=pltpu.CompilerParams(
            dimension_semantics=("arbitrary", "arbitrary", "arbitrary")),
    )(mx, h, deg_raw)


def _mm2_body(m_ref, t_ref, s_ref, o_ref):
    k = pl.program_id(2)
    nk = pl.num_programs(2)
    part = lax.dot_general(m_ref[0], t_ref[0].astype(jnp.bfloat16),
                           (((0,), (0,)), ((), ())),
                           preferred_element_type=jnp.float32)

    @pl.when(k == 0)
    def _():
        o_ref[...] = jnp.zeros_like(o_ref)

    o_ref[...] += part[None]

    @pl.when(k == nk - 1)
    def _():
        sc = _dinv(s_ref[0, 0], None)
        o_ref[...] *= sc[None, :, None]


def _mm2(mx, base, t, s_raw):
    grid = (NC, N // _JB, N // _KB)
    return pl.pallas_call(
        _mm2_body,
        grid=grid,
        in_specs=[
            pl.BlockSpec((1, _KB, _JB), lambda c, j, k: (base + c, k, j)),
            pl.BlockSpec((1, _KB, F_IN), lambda c, j, k: (c, k, 0)),
            pl.BlockSpec((1, 1, _JB), lambda c, j, k: (c, 0, j)),
        ],
        out_specs=pl.BlockSpec((1, _JB, F_IN), lambda c, j, k: (c, j, 0)),
        out_shape=jax.ShapeDtypeStruct((NC, N, F_IN), jnp.float32),
        compiler_params=pltpu.CompilerParams(
            dimension_semantics=("arbitrary", "arbitrary", "arbitrary")),
    )(mx, t, s_raw)


def _mm3_body(m_ref, t_ref, s_ref, w_ref, b_ref, o_ref, acc):
    k = pl.program_id(2)
    nk = pl.num_programs(2)
    part = lax.dot_general(m_ref[0], t_ref[0].astype(jnp.bfloat16),
                           (((0,), (0,)), ((), ())),
                           preferred_element_type=jnp.float32)

    @pl.when(k == 0)
    def _():
        acc[...] = jnp.zeros_like(acc)

    acc[...] += part

    @pl.when(k == nk - 1)
    def _():
        sc = _dinv(s_ref[0, 0], None)
        agg = acc[...] * sc[:, None]
        res = jnp.dot(agg, w_ref[...], preferred_element_type=jnp.float32)
        o_ref[...] = jnp.maximum(res + b_ref[...], 0.0)[None]


def _mm3(mx, base, t, s_raw, gcn_w, gcn_b):
    grid = (NC, N // _JB, N // _KB)
    return pl.pallas_call(
        _mm3_body,
        grid=grid,
        in_specs=[
            pl.BlockSpec((1, _KB, _JB), lambda c, j, k: (base + c, k, j)),
            pl.BlockSpec((1, _KB, F_IN), lambda c, j, k: (c, k, 0)),
            pl.BlockSpec((1, 1, _JB), lambda c, j, k: (c, 0, j)),
            pl.BlockSpec((F_IN, F_OUT), lambda c, j, k: (0, 0)),
            pl.BlockSpec((1, F_OUT), lambda c, j, k: (0, 0)),
        ],
        out_specs=pl.BlockSpec((1, _JB, F_OUT), lambda c, j, k: (c, j, 0)),
        out_shape=jax.ShapeDtypeStruct((NC, N, F_OUT), jnp.float32),
        scratch_shapes=[pltpu.VMEM((_JB, F_IN), jnp.float32)],
        compiler_params=pltpu.CompilerParams(
            dimension_semantics=("arbitrary", "arbitrary", "arbitrary")),
    )(mx, t, s_raw, gcn_w, gcn_b)


def _head_body(x_ref, w1_ref, b1_ref, w2_ref, b2_ref, o_ref):
    xc = jnp.concatenate([x_ref[0], x_ref[1]], axis=1)
    x1 = jnp.dot(xc, w1_ref[...], preferred_element_type=jnp.float32)
    x1 = jnp.maximum(x1 + b1_ref[...], 0.0)
    y = jnp.dot(x1, w2_ref[...], preferred_element_type=jnp.float32)
    o_ref[...] = y + b2_ref[...]


def _head(x, w1, b1, w2, b2):
    grid = (N // _JB,)
    return pl.pallas_call(
        _head_body,
        grid=grid,
        in_specs=[
            pl.BlockSpec((NC, _JB, F_OUT), lambda r: (0, r, 0)),
            pl.BlockSpec((NC * F_OUT, F_OUT), lambda r: (0, 0)),
            pl.BlockSpec((1, F_OUT), lambda r: (0, 0)),
            pl.BlockSpec((F_OUT, NCLS), lambda r: (0, 0)),
            pl.BlockSpec((1, NCLS), lambda r: (0, 0)),
        ],
        out_specs=pl.BlockSpec((_JB, NCLS), lambda r: (r, 0)),
        out_shape=jax.ShapeDtypeStruct((N, NCLS), jnp.float32),
        compiler_params=pltpu.CompilerParams(
            dimension_semantics=("arbitrary",)),
    )(x, w1, b1, w2, b2)


def _dense_pipeline(a, h, w_l0c1, w_l0c2, w_l1c1,
                    gcn_W, gcn_b, lin1_W, lin1_b, lin2_W, lin2_b):
    mx, u = _combine(w_l0c1, w_l0c2, w_l1c1, a)

    s0 = _mvt(mx, NC, u, u, _id2)
    s1 = _mvt(mx, 2 * NC, s0, s0, _maskf)
    y1 = _mvn(mx, 2 * NC, s1, s1, _dinv)
    y2 = _mvn(mx, NC, s0, y1, _dinv_mul)
    deg = _mvn(mx, 0, y2, y2, _id2)

    t1 = _mm1(mx, h, deg)
    t2 = _mm2(mx, NC, t1, s0)
    x = _mm3(mx, 2 * NC, t2, s1, gcn_W, gcn_b.reshape(1, F_OUT))
    return _head(x, lin1_W, lin1_b.reshape(1, F_OUT),
                 lin2_W, lin2_b.reshape(1, NCLS))


def kernel(edge_index, edge_value, h, w_l0c1, w_l0c2, w_l1c1,
           gcn_W, gcn_b, lin1_W, lin1_b, lin2_W, lin2_b):
    src = edge_index[:, 0, :].astype(jnp.int32)
    dst = edge_index[:, 1, :].astype(jnp.int32)
    a = _build_adjacency(src, dst, edge_value)
    a = a.reshape(NT, N, N)
    return _dense_pipeline(a, h, w_l0c1, w_l0c2, w_l1c1,
                           gcn_W, gcn_b, lin1_W, lin1_b, lin2_W, lin2_b)
